# baseline (speedup 1.0000x reference)
import jax
import jax.numpy as jnp
from jax import lax
from jax.experimental import pallas as pl
from jax.experimental.pallas import tpu as pltpu
from jax.experimental.pallas import tpu_sc as plsc

B = 4
N = 4096
L = 16
NC = 2

SRC = 512
MTC = N - SRC
YC = 512
NGY = YC // L
NXB = SRC // L
TOFF = 0
SOFF = B * 3 * N
ROWSZ = B * N

TN = 1024
NT = N // TN
KP = 128
MS = 512
NM = MTC // MS


def _rbf16(v):
    c = v * jnp.float32(65537.0)
    return c - (c - v)


def _sc_body(p_hbm, out_hbm, yx, yy, yz, y2, sx, sy, sz, s2, macc, colp):
    s = lax.axis_index("s")
    c = lax.axis_index("c")
    wid = s * NC + c
    b = wid % B
    w2 = wid // B

    tbase = b * 3 * N + w2 * YC
    sbase = SOFF + b * 3 * SRC
    pltpu.sync_copy(p_hbm.at[pl.ds(tbase, YC)], yx)
    pltpu.sync_copy(p_hbm.at[pl.ds(tbase + N, YC)], yy)
    pltpu.sync_copy(p_hbm.at[pl.ds(tbase + 2 * N, YC)], yz)
    pltpu.sync_copy(p_hbm.at[pl.ds(sbase, SRC)], sx)
    pltpu.sync_copy(p_hbm.at[pl.ds(sbase + SRC, SRC)], sy)
    pltpu.sync_copy(p_hbm.at[pl.ds(sbase + 2 * SRC, SRC)], sz)

    inf16 = jnp.full((L,), jnp.float32(jnp.inf), jnp.float32)

    def pry(g, carry):
        sl = pl.ds(g * L, L)
        ax, ay, az = yx[sl], yy[sl], yz[sl]
        y2[sl] = ax * ax + ay * ay + az * az
        yx[sl] = _rbf16(ax)
        yy[sl] = _rbf16(ay)
        yz[sl] = _rbf16(az)
        macc[sl] = inf16
        return carry

    lax.fori_loop(0, NGY, pry, 0)

    def prs(g, carry):
        sl = pl.ds(g * L, L)
        ax, ay, az = sx[sl], sy[sl], sz[sl]
        s2[sl] = ax * ax + ay * ay + az * az
        sx[sl] = _rbf16(ax) * -2.0
        sy[sl] = _rbf16(ay) * -2.0
        sz[sl] = _rbf16(az) * -2.0
        return carry

    lax.fori_loop(0, NXB, prs, 0)

    JS = 4
    iota = jnp.arange(L, dtype=jnp.int32)

    def xblk(xb, carry):
        xsl = pl.ds(xb * L, L)
        bx, by_, bz, b2 = sx[xsl], sy[xsl], sz[xsl], s2[xsl]
        acc = None
        for js in range(L // JS):
            vsx = [bx[js * JS + j] for j in range(JS)]
            vsy = [by_[js * JS + j] for j in range(JS)]
            vsz = [bz[js * JS + j] for j in range(JS)]
            vs2 = [b2[js * JS + j] for j in range(JS)]

            def gbody(g, tj):
                sl = pl.ds(g * L, L)
                vx, vy, vz, v2 = yx[sl], yy[sl], yz[sl], y2[sl]
                m = macc[sl]
                out = []
                for j in range(JS):
                    p = vx * vsx[j] + vy * vsy[j] + vz * vsz[j]
                    m = jnp.minimum(m, p + vs2[j])
                    out.append(jnp.minimum(tj[j], p + v2))
                macc[sl] = m
                return tuple(out)

            tj = lax.fori_loop(0, NGY, gbody, (inf16,) * JS, unroll=2)

            for j in range(JS):
                v = tj[j]
                for sh in (8, 4, 2, 1):
                    v = jnp.minimum(v, jnp.take(v, iota ^ sh))
                jj = js * JS + j
                acc = v if acc is None else jnp.where(iota == jj, v, acc)
        colp[pl.ds(xb * L, L)] = acc
        return carry

    lax.fori_loop(0, NXB, xblk, 0)

    def rfin(g, carry):
        sl = pl.ds(g * L, L)
        macc[sl] = macc[sl] + y2[sl]
        return carry

    lax.fori_loop(0, NGY, rfin, 0)
    pltpu.sync_copy(macc, out_hbm.at[pl.ds(b * N + w2 * YC, YC)])
    pltpu.sync_copy(colp, out_hbm.at[pl.ds(ROWSZ + (b * 8 + w2) * SRC, SRC)])


_sc_mins = pl.kernel(
    _sc_body,
    out_type=jax.ShapeDtypeStruct((ROWSZ + B * 8 * SRC,), jnp.float32),
    mesh=plsc.VectorSubcoreMesh(core_axis_name="c", subcore_axis_name="s"),
    scratch_types=[
        pltpu.VMEM((YC,), jnp.float32),
        pltpu.VMEM((YC,), jnp.float32),
        pltpu.VMEM((YC,), jnp.float32),
        pltpu.VMEM((YC,), jnp.float32),
        pltpu.VMEM((SRC,), jnp.float32),
        pltpu.VMEM((SRC,), jnp.float32),
        pltpu.VMEM((SRC,), jnp.float32),
        pltpu.VMEM((SRC,), jnp.float32),
        pltpu.VMEM((YC,), jnp.float32),
        pltpu.VMEM((SRC,), jnp.float32),
    ],
)


def _tc_body(t3_ref, s3T_ref, tb_ref, sb_ref, row_ref, col_ref, ca_ref, cb_ref,
             racc_ref):
    b = pl.program_id(0)
    nt = pl.program_id(1)
    tblk = t3_ref[0]
    t2 = jnp.sum(tblk * tblk, axis=1)

    @pl.when(nt == 0)
    def _():
        col_ref[...] = jnp.full((8, MTC), jnp.float32(jnp.inf), jnp.float32)

    for mi in range(NM):
        cref = ca_ref if mi % 2 == 0 else cb_ref
        msl = pl.ds(mi * MS, MS)
        sblk = s3T_ref[0, :, msl]
        s2 = jnp.sum(sblk * sblk, axis=0)
        cref[...] = jnp.dot(
            tb_ref[0], sb_ref[0, :, msl], preferred_element_type=jnp.float32
        )

        rowpath = s2[None, :] + cref[...]
        rparts = [rowpath[:, 128 * i : 128 * (i + 1)] for i in range(MS // 128)]
        rp = rparts[0]
        for piece in rparts[1:]:
            rp = jnp.minimum(rp, piece)
        if mi == 0:
            racc_ref[...] = rp
        else:
            racc_ref[...] = jnp.minimum(racc_ref[...], rp)

        colpath = t2[:, None] + cref[...]
        parts = [colpath[8 * i : 8 * i + 8, :] for i in range(TN // 8)]
        while len(parts) > 1:
            parts = [
                jnp.minimum(parts[2 * i], parts[2 * i + 1])
                for i in range(len(parts) // 2)
            ]
        col_ref[:, msl] = jnp.minimum(col_ref[:, msl], parts[0])

    rowmin = t2 + jnp.min(racc_ref[...], axis=1)
    row_ref[pl.ds(b, 1), pl.ds(nt * TN, TN)] = rowmin.reshape(1, TN)


_tc_call = pl.pallas_call(
    _tc_body,
    grid=(B, NT),
    in_specs=[
        pl.BlockSpec((1, TN, 3), lambda b, nt: (b, nt, 0)),
        pl.BlockSpec((1, 3, MTC), lambda b, nt: (b, 0, 0)),
        pl.BlockSpec((1, TN, KP), lambda b, nt: (b, nt, 0)),
        pl.BlockSpec((1, KP, MTC), lambda b, nt: (b, 0, 0)),
    ],
    out_specs=[
        pl.BlockSpec((B, N), lambda b, nt: (0, 0)),
        pl.BlockSpec((8, MTC), lambda b, nt: (b, 0)),
    ],
    out_shape=[
        jax.ShapeDtypeStruct((B, N), jnp.float32),
        jax.ShapeDtypeStruct((8 * B, MTC), jnp.float32),
    ],
    scratch_shapes=[
        pltpu.VMEM((TN, MS), jnp.float32),
        pltpu.VMEM((TN, MS), jnp.float32),
        pltpu.VMEM((TN, 128), jnp.float32),
    ],
)


def _fin_body(row_ref, col_ref, s3T_ref, scrow_ref, sccol_ref, o_ref):
    s2 = jnp.sum(s3T_ref[...] * s3T_ref[...], axis=1)
    rowd2 = jnp.minimum(row_ref[...], scrow_ref[...])
    tccol = jnp.min(col_ref[...].reshape(B, 8, MTC), axis=1) + s2[:, :MTC]
    sccol = jnp.min(sccol_ref[...], axis=1) + s2[:, MTC:]
    tot = (
        jnp.sum(jnp.sqrt(jnp.maximum(rowd2, 0.0)))
        + jnp.sum(jnp.sqrt(jnp.maximum(tccol, 0.0)))
        + jnp.sum(jnp.sqrt(jnp.maximum(sccol, 0.0)))
    )
    o_ref[0, 0] = tot / jnp.float32(2 * B * N)


_finish = pl.pallas_call(
    _fin_body,
    out_shape=jax.ShapeDtypeStruct((1, 1), jnp.float32),
    out_specs=pl.BlockSpec(memory_space=pltpu.SMEM),
)


def kernel(template, source):
    t3T = jnp.transpose(template, (0, 2, 1))
    s3T = jnp.transpose(source, (0, 2, 1))
    p_sc = jnp.concatenate(
        [t3T.reshape(-1), s3T[:, :, MTC:].reshape(-1)]
    )
    sc_out = _sc_mins(p_sc)

    tb = jnp.pad(
        template.astype(jnp.bfloat16) * jnp.bfloat16(-2.0),
        ((0, 0), (0, 0), (0, KP - 3)),
    )
    sb = jnp.transpose(
        jnp.pad(source.astype(jnp.bfloat16), ((0, 0), (0, 0), (0, KP - 3))),
        (0, 2, 1),
    )[:, :, :MTC]
    row, col = _tc_call(template, s3T[:, :, :MTC], tb, sb)

    sc_row = sc_out[:ROWSZ].reshape(B, N)
    sc_colp = sc_out[ROWSZ:].reshape(B, 8, SRC)
    loss = _finish(row, col, s3T, sc_row, sc_colp)
    return loss[0, 0]


# speedup vs baseline: 1.0441x; 1.0441x over previous
import jax
import jax.numpy as jnp
from jax import lax
from jax.experimental import pallas as pl
from jax.experimental.pallas import tpu as pltpu
from jax.experimental.pallas import tpu_sc as plsc

B = 4
N = 4096
L = 16
NC = 2

SRC = 256
MTC = N - SRC
YC = 512
NGY = YC // L
NXB = SRC // L
TOFF = 0
SOFF = B * 3 * N
ROWSZ = B * N

TN = 1024
NT = N // TN
KP = 128
MS = 384
NM = MTC // MS


def _rbf16(v):
    c = v * jnp.float32(65537.0)
    return c - (c - v)


def _sc_body(p_hbm, out_hbm, yx, yy, yz, y2, sx, sy, sz, s2, macc, colp):
    s = lax.axis_index("s")
    c = lax.axis_index("c")
    wid = s * NC + c
    b = wid % B
    w2 = wid // B

    tbase = b * 3 * N + w2 * YC
    sbase = SOFF + b * 3 * SRC
    pltpu.sync_copy(p_hbm.at[pl.ds(tbase, YC)], yx)
    pltpu.sync_copy(p_hbm.at[pl.ds(tbase + N, YC)], yy)
    pltpu.sync_copy(p_hbm.at[pl.ds(tbase + 2 * N, YC)], yz)
    pltpu.sync_copy(p_hbm.at[pl.ds(sbase, SRC)], sx)
    pltpu.sync_copy(p_hbm.at[pl.ds(sbase + SRC, SRC)], sy)
    pltpu.sync_copy(p_hbm.at[pl.ds(sbase + 2 * SRC, SRC)], sz)

    inf16 = jnp.full((L,), jnp.float32(jnp.inf), jnp.float32)

    def pry(g, carry):
        sl = pl.ds(g * L, L)
        ax, ay, az = yx[sl], yy[sl], yz[sl]
        y2[sl] = ax * ax + ay * ay + az * az
        yx[sl] = _rbf16(ax)
        yy[sl] = _rbf16(ay)
        yz[sl] = _rbf16(az)
        macc[sl] = inf16
        return carry

    lax.fori_loop(0, NGY, pry, 0)

    def prs(g, carry):
        sl = pl.ds(g * L, L)
        ax, ay, az = sx[sl], sy[sl], sz[sl]
        s2[sl] = ax * ax + ay * ay + az * az
        sx[sl] = _rbf16(ax) * -2.0
        sy[sl] = _rbf16(ay) * -2.0
        sz[sl] = _rbf16(az) * -2.0
        return carry

    lax.fori_loop(0, NXB, prs, 0)

    JS = 4
    iota = jnp.arange(L, dtype=jnp.int32)

    def xblk(xb, carry):
        xsl = pl.ds(xb * L, L)
        bx, by_, bz, b2 = sx[xsl], sy[xsl], sz[xsl], s2[xsl]
        acc = None
        for js in range(L // JS):
            vsx = [bx[js * JS + j] for j in range(JS)]
            vsy = [by_[js * JS + j] for j in range(JS)]
            vsz = [bz[js * JS + j] for j in range(JS)]
            vs2 = [b2[js * JS + j] for j in range(JS)]

            def gbody(g, tj):
                sl = pl.ds(g * L, L)
                vx, vy, vz, v2 = yx[sl], yy[sl], yz[sl], y2[sl]
                m = macc[sl]
                out = []
                for j in range(JS):
                    p = vx * vsx[j] + vy * vsy[j] + vz * vsz[j]
                    m = jnp.minimum(m, p + vs2[j])
                    out.append(jnp.minimum(tj[j], p + v2))
                macc[sl] = m
                return tuple(out)

            tj = lax.fori_loop(0, NGY, gbody, (inf16,) * JS, unroll=2)

            for j in range(JS):
                v = tj[j]
                for sh in (8, 4, 2, 1):
                    v = jnp.minimum(v, jnp.take(v, iota ^ sh))
                jj = js * JS + j
                acc = v if acc is None else jnp.where(iota == jj, v, acc)
        colp[pl.ds(xb * L, L)] = acc
        return carry

    lax.fori_loop(0, NXB, xblk, 0)

    def rfin(g, carry):
        sl = pl.ds(g * L, L)
        macc[sl] = macc[sl] + y2[sl]
        return carry

    lax.fori_loop(0, NGY, rfin, 0)
    pltpu.sync_copy(macc, out_hbm.at[pl.ds(b * N + w2 * YC, YC)])
    pltpu.sync_copy(colp, out_hbm.at[pl.ds(ROWSZ + (b * 8 + w2) * SRC, SRC)])


_sc_mins = pl.kernel(
    _sc_body,
    out_type=jax.ShapeDtypeStruct((ROWSZ + B * 8 * SRC,), jnp.float32),
    mesh=plsc.VectorSubcoreMesh(core_axis_name="c", subcore_axis_name="s"),
    scratch_types=[
        pltpu.VMEM((YC,), jnp.float32),
        pltpu.VMEM((YC,), jnp.float32),
        pltpu.VMEM((YC,), jnp.float32),
        pltpu.VMEM((YC,), jnp.float32),
        pltpu.VMEM((SRC,), jnp.float32),
        pltpu.VMEM((SRC,), jnp.float32),
        pltpu.VMEM((SRC,), jnp.float32),
        pltpu.VMEM((SRC,), jnp.float32),
        pltpu.VMEM((YC,), jnp.float32),
        pltpu.VMEM((SRC,), jnp.float32),
    ],
)


def _tc_body(t3_ref, s3T_ref, tb_ref, sb_ref, row_ref, col_ref, ca_ref, cb_ref,
             racc_ref):
    b = pl.program_id(0)
    nt = pl.program_id(1)
    tblk = t3_ref[0]
    t2 = jnp.sum(tblk * tblk, axis=1)

    @pl.when(nt == 0)
    def _():
        col_ref[...] = jnp.full((8, MTC), jnp.float32(jnp.inf), jnp.float32)

    for mi in range(NM):
        cref = ca_ref if mi % 2 == 0 else cb_ref
        msl = pl.ds(mi * MS, MS)
        sblk = s3T_ref[0, :, msl]
        s2 = jnp.sum(sblk * sblk, axis=0)
        cref[...] = jnp.dot(
            tb_ref[0], sb_ref[0, :, msl], preferred_element_type=jnp.float32
        )

        rowpath = s2[None, :] + cref[...]
        rparts = [rowpath[:, 128 * i : 128 * (i + 1)] for i in range(MS // 128)]
        rp = rparts[0]
        for piece in rparts[1:]:
            rp = jnp.minimum(rp, piece)
        if mi == 0:
            racc_ref[...] = rp
        else:
            racc_ref[...] = jnp.minimum(racc_ref[...], rp)

        colpath = t2[:, None] + cref[...]
        parts = [colpath[8 * i : 8 * i + 8, :] for i in range(TN // 8)]
        while len(parts) > 1:
            parts = [
                jnp.minimum(parts[2 * i], parts[2 * i + 1])
                for i in range(len(parts) // 2)
            ]
        col_ref[:, msl] = jnp.minimum(col_ref[:, msl], parts[0])

    rowmin = t2 + jnp.min(racc_ref[...], axis=1)
    row_ref[pl.ds(b, 1), pl.ds(nt * TN, TN)] = rowmin.reshape(1, TN)


_tc_call = pl.pallas_call(
    _tc_body,
    grid=(B, NT),
    in_specs=[
        pl.BlockSpec((1, TN, 3), lambda b, nt: (b, nt, 0)),
        pl.BlockSpec((1, 3, MTC), lambda b, nt: (b, 0, 0)),
        pl.BlockSpec((1, TN, KP), lambda b, nt: (b, nt, 0)),
        pl.BlockSpec((1, KP, MTC), lambda b, nt: (b, 0, 0)),
    ],
    out_specs=[
        pl.BlockSpec((B, N), lambda b, nt: (0, 0)),
        pl.BlockSpec((8, MTC), lambda b, nt: (b, 0)),
    ],
    out_shape=[
        jax.ShapeDtypeStruct((B, N), jnp.float32),
        jax.ShapeDtypeStruct((8 * B, MTC), jnp.float32),
    ],
    scratch_shapes=[
        pltpu.VMEM((TN, MS), jnp.float32),
        pltpu.VMEM((TN, MS), jnp.float32),
        pltpu.VMEM((TN, 128), jnp.float32),
    ],
)


def _fin_body(row_ref, col_ref, s3T_ref, scrow_ref, sccol_ref, o_ref):
    s2 = jnp.sum(s3T_ref[...] * s3T_ref[...], axis=1)
    rowd2 = jnp.minimum(row_ref[...], scrow_ref[...])
    tccol = jnp.min(col_ref[...].reshape(B, 8, MTC), axis=1) + s2[:, :MTC]
    sccol = jnp.min(sccol_ref[...], axis=1) + s2[:, MTC:]
    tot = (
        jnp.sum(jnp.sqrt(jnp.maximum(rowd2, 0.0)))
        + jnp.sum(jnp.sqrt(jnp.maximum(tccol, 0.0)))
        + jnp.sum(jnp.sqrt(jnp.maximum(sccol, 0.0)))
    )
    o_ref[0, 0] = tot / jnp.float32(2 * B * N)


_finish = pl.pallas_call(
    _fin_body,
    out_shape=jax.ShapeDtypeStruct((1, 1), jnp.float32),
    out_specs=pl.BlockSpec(memory_space=pltpu.SMEM),
)


def kernel(template, source):
    t3T = jnp.transpose(template, (0, 2, 1))
    s3T = jnp.transpose(source, (0, 2, 1))
    p_sc = jnp.concatenate(
        [t3T.reshape(-1), s3T[:, :, MTC:].reshape(-1)]
    )
    sc_out = _sc_mins(p_sc)

    tb = jnp.pad(
        template.astype(jnp.bfloat16) * jnp.bfloat16(-2.0),
        ((0, 0), (0, 0), (0, KP - 3)),
    )
    sb = jnp.transpose(
        jnp.pad(source.astype(jnp.bfloat16), ((0, 0), (0, 0), (0, KP - 3))),
        (0, 2, 1),
    )[:, :, :MTC]
    row, col = _tc_call(template, s3T[:, :, :MTC], tb, sb)

    sc_row = sc_out[:ROWSZ].reshape(B, N)
    sc_colp = sc_out[ROWSZ:].reshape(B, 8, SRC)
    loss = _finish(row, col, s3T, sc_row, sc_colp)
    return loss[0, 0]
